# Initial kernel scaffold; baseline (speedup 1.0000x reference)
import jax
import jax.numpy as jnp
from jax.experimental import pallas as pl


def kernel(input, W1, W2):
    raise NotImplementedError("write your pallas kernel here")


# speedup vs baseline: 2.5583x; 2.5583x over previous
import functools

import jax
import jax.numpy as jnp
from jax.experimental import pallas as pl

_MODEL_DIM = 4096
_NUM_EXPERTS = 64
_HIDDEN = _NUM_EXPERTS * 8
_NUM_TOKENS = 16384
_TM = 512


def _router_tile(x_ref, w1_ref, w2_ref, dense_ref, logits_ref):
    x = x_ref[...]
    h = jnp.tanh(
        jax.lax.dot_general(
            x, w1_ref[...], (((1,), (1,)), ((), ())),
            preferred_element_type=jnp.float32,
        )
    )
    logits = jax.lax.dot_general(
        h, w2_ref[...], (((1,), (1,)), ((), ())),
        preferred_element_type=jnp.float32,
    )
    mean = jnp.mean(logits, axis=1, keepdims=True)
    var = jnp.sum((logits - mean) ** 2, axis=1, keepdims=True) / (
        _NUM_EXPERTS - 1
    )
    logits = logits / jnp.sqrt(var)

    col = jax.lax.broadcasted_iota(jnp.int32, logits.shape, 1)
    m1 = jnp.max(logits, axis=1, keepdims=True)
    i1 = jnp.argmax(logits, axis=1)[:, None]
    masked = jnp.where(col == i1, -jnp.inf, logits)
    m2 = jnp.max(masked, axis=1, keepdims=True)
    i2 = jnp.argmax(masked, axis=1)[:, None]

    e = jnp.exp(m2 - m1)
    s = 1.0 + e
    g1 = 1.0 / s
    g2 = e / s

    dense_ref[...] = jnp.where(col == i1, g1, 0.0) + jnp.where(col == i2, g2, 0.0)
    logits_ref[...] = logits


@functools.partial(jax.jit, static_argnames=())
def kernel(input, W1, W2):
    x = input.astype(jnp.float32)
    n_tokens = x.shape[0]
    grid = (n_tokens // _TM,)
    dense, logits = pl.pallas_call(
        _router_tile,
        grid=grid,
        in_specs=[
            pl.BlockSpec((_TM, _MODEL_DIM), lambda i: (i, 0)),
            pl.BlockSpec((_HIDDEN, _MODEL_DIM), lambda i: (0, 0)),
            pl.BlockSpec((_NUM_EXPERTS, _HIDDEN), lambda i: (0, 0)),
        ],
        out_specs=[
            pl.BlockSpec((_TM, _NUM_EXPERTS), lambda i: (i, 0)),
            pl.BlockSpec((_TM, _NUM_EXPERTS), lambda i: (i, 0)),
        ],
        out_shape=[
            jax.ShapeDtypeStruct((n_tokens, _NUM_EXPERTS), jnp.float32),
            jax.ShapeDtypeStruct((n_tokens, _NUM_EXPERTS), jnp.float32),
        ],
    )(x, W1, W2)
    return (dense, logits)


# speedup vs baseline: 2.8288x; 1.1057x over previous
import functools

import jax
import jax.numpy as jnp
from jax.experimental import pallas as pl

_MODEL_DIM = 4096
_NUM_EXPERTS = 64
_HIDDEN = _NUM_EXPERTS * 8
_NUM_TOKENS = 16384
_TM = 1024


def _router_tile(x_ref, w1_ref, w2_ref, dense_ref, logits_ref):
    x = x_ref[...]
    h = jnp.tanh(
        jax.lax.dot_general(
            x, w1_ref[...], (((1,), (1,)), ((), ())),
            preferred_element_type=jnp.float32,
        )
    )
    logits = jax.lax.dot_general(
        h, w2_ref[...], (((1,), (1,)), ((), ())),
        preferred_element_type=jnp.float32,
    )
    mean = jnp.mean(logits, axis=1, keepdims=True)
    var = jnp.sum((logits - mean) ** 2, axis=1, keepdims=True) / (
        _NUM_EXPERTS - 1
    )
    logits = logits / jnp.sqrt(var)

    col = jax.lax.broadcasted_iota(jnp.int32, logits.shape, 1)
    m1 = jnp.max(logits, axis=1, keepdims=True)
    i1 = jnp.argmax(logits, axis=1)[:, None]
    masked = jnp.where(col == i1, -jnp.inf, logits)
    m2 = jnp.max(masked, axis=1, keepdims=True)
    i2 = jnp.argmax(masked, axis=1)[:, None]

    e = jnp.exp(m2 - m1)
    s = 1.0 + e
    g1 = 1.0 / s
    g2 = e / s

    dense_ref[...] = jnp.where(col == i1, g1, 0.0) + jnp.where(col == i2, g2, 0.0)
    logits_ref[...] = logits


@functools.partial(jax.jit, static_argnames=())
def kernel(input, W1, W2):
    x = input.astype(jnp.float32)
    n_tokens = x.shape[0]
    grid = (n_tokens // _TM,)
    dense, logits = pl.pallas_call(
        _router_tile,
        grid=grid,
        in_specs=[
            pl.BlockSpec((_TM, _MODEL_DIM), lambda i: (i, 0)),
            pl.BlockSpec((_HIDDEN, _MODEL_DIM), lambda i: (0, 0)),
            pl.BlockSpec((_NUM_EXPERTS, _HIDDEN), lambda i: (0, 0)),
        ],
        out_specs=[
            pl.BlockSpec((_TM, _NUM_EXPERTS), lambda i: (i, 0)),
            pl.BlockSpec((_TM, _NUM_EXPERTS), lambda i: (i, 0)),
        ],
        out_shape=[
            jax.ShapeDtypeStruct((n_tokens, _NUM_EXPERTS), jnp.float32),
            jax.ShapeDtypeStruct((n_tokens, _NUM_EXPERTS), jnp.float32),
        ],
    )(x, W1, W2)
    return (dense, logits)
